# baseline (speedup 1.0000x reference)
import functools

import jax
import jax.numpy as jnp
from jax.experimental import pallas as pl
from jax.experimental.pallas import tpu as pltpu

CHUNK = 64
BF = jnp.bfloat16


def _mm(a, b):
    return jax.lax.dot_general(a.astype(BF), b.astype(BF),
                               (((1,), (0,)), ((), ())),
                               preferred_element_type=jnp.float32)


def _mmT(a, b):
    return jax.lax.dot_general(a.astype(BF), b.astype(BF),
                               (((1,), (1,)), ((), ())),
                               preferred_element_type=jnp.float32)


def _mTm(a, b):
    return jax.lax.dot_general(a.astype(BF), b.astype(BF),
                               (((0,), (0,)), ((), ())),
                               preferred_element_type=jnp.float32)


def _delta_kernel(x_ref, w_ref, b_ref, o_ref, scr_ref, cf_ref, bs_ref,
                  s_ref, *, L, D, hd):
    C = CHUNK
    n_chunks = L // C
    w = w_ref[0]
    bias = b_ref[0]

    def project(r0, nrows):
        xs = x_ref[0, pl.ds(r0, nrows), :]
        scr_ref[pl.ds(r0, nrows), :] = (_mmT(xs, w) + bias).astype(BF)

    row = jax.lax.broadcasted_iota(jnp.int32, (C, C), 0)
    col = jax.lax.broadcasted_iota(jnp.int32, (C, C), 1)
    strict = row > col
    incl = row >= col
    eye = jnp.where(row == col, 1.0, 0.0).astype(jnp.float32)
    scale = BF(hd ** -0.5)

    def pre_pair(i0):
        J = (0, 1)
        pcs = [scr_ref[pl.ds((i0 + j) * C, C), :] for j in J]
        qs = [pc[:, 0:hd] * scale for pc in pcs]
        ks = [pc[:, hd:2 * hd] for pc in pcs]
        betas = [jax.nn.sigmoid(pc[:, 3 * hd:4 * hd].astype(jnp.float32)).astype(BF)
                 for pc in pcs]
        kbs = [ks[j] * betas[j] for j in J]
        vbs = [pcs[j][:, 2 * hd:3 * hd] * betas[j] for j in J]
        ABs = [_mmT(jnp.concatenate([kbs[j], qs[j]], axis=0), ks[j]) for j in J]
        As = [jnp.where(strict, AB[:C], 0.0) for AB in ABs]
        attns = [jnp.where(incl, AB[C:], 0.0) for AB in ABs]
        Xs = [eye - A for A in As]
        Qs = [_mm(A, A) for A in As]
        for _ in range(4):
            RXs = [_mm(Qs[j], Xs[j]) for j in J]
            RQs = [_mm(Qs[j], Qs[j]) for j in J]
            Xs = [Xs[j] + RXs[j] for j in J]
            Qs = RQs
        Xs = [Xs[j] + _mm(Qs[j], Xs[j]) for j in J]
        u0s = [_mm(Xs[j], vbs[j]) for j in J]
        ws = [_mm(Xs[j], kbs[j]) for j in J]
        AUs = [_mm(attns[j], u0s[j]) for j in J]
        AWs = [_mm(attns[j], ws[j]) for j in J]
        KUs = [_mTm(ks[j], u0s[j]) for j in J]
        KWs = [_mTm(ks[j], ws[j]) for j in J]
        for j in J:
            cf = jnp.concatenate([qs[j] - AWs[j], eye - KWs[j]], axis=0)
            bs = jnp.concatenate([AUs[j], KUs[j]], axis=0)
            cf_ref[pl.ds((i0 + j) * 2 * C, 2 * C), :] = cf.astype(BF)
            bs_ref[pl.ds((i0 + j) * 2 * C, 2 * C), :] = bs.astype(BF)

    def scan_chunk(i):
        cf = cf_ref[pl.ds(i * 2 * C, 2 * C), :]
        bs = bs_ref[pl.ds(i * 2 * C, 2 * C), :]
        Z = _mm(cf, s_ref[C:2 * C, :]) + bs.astype(jnp.float32)
        o_ref[0, pl.ds(i * C, C), 0, 0, :] = Z[:C].astype(BF)
        s_ref[...] = Z

    n_pairs = n_chunks // 2

    s_ref[...] = jnp.zeros((2 * C, hd), jnp.float32)
    project(0, 4 * C)
    pre_pair(0)

    def body(jj, carry):
        scan_chunk(2 * jj)
        scan_chunk(2 * jj + 1)
        pre_pair(2 * jj + 2)
        project(jnp.minimum(2 * jj + 4, n_chunks - 2) * C, 2 * C)
        return carry

    jax.lax.fori_loop(0, n_pairs - 1, body, jnp.float32(0.0))
    scan_chunk(n_chunks - 2)
    scan_chunk(n_chunks - 1)


def _out_kernel(o_ref, w_ref, b_ref, y_ref):
    y_ref[...] = _mmT(o_ref[...], w_ref[...]) + b_ref[...]


def kernel(hidden_states, Wq, bq, Wk, bk, Wv, bv, Wb, bb, Wo, bo):
    x = hidden_states
    B, L, D = x.shape
    H = Wb.shape[0]
    hd = D // H
    BH = B * H

    Wq_r = Wq.reshape(H, hd, D)
    Wk_r = Wk.reshape(H, hd, D)
    Wv_r = Wv.reshape(H, hd, D)
    Wb_r = jnp.broadcast_to(Wb[:, None, :], (H, hd, D))
    W_all = jnp.concatenate([Wq_r, Wk_r, Wv_r, Wb_r], axis=1).astype(BF)
    b_all = jnp.concatenate(
        [bq.reshape(H, hd), bk.reshape(H, hd), bv.reshape(H, hd),
         jnp.broadcast_to(bb[:, None], (H, hd))], axis=1).reshape(H, 1, 4 * hd)

    o_heads = pl.pallas_call(
        functools.partial(_delta_kernel, L=L, D=D, hd=hd),
        grid=(BH,),
        in_specs=[
            pl.BlockSpec((1, L, D), lambda i: (i // H, 0, 0)),
            pl.BlockSpec((1, 4 * hd, D), lambda i: (i % H, 0, 0)),
            pl.BlockSpec((1, 1, 4 * hd), lambda i: (i % H, 0, 0)),
        ],
        out_specs=pl.BlockSpec((1, L, 1, 1, hd), lambda i: (i // H, 0, i % H, 0, 0)),
        out_shape=jax.ShapeDtypeStruct((B, L, H, 1, hd), BF),
        scratch_shapes=[
            pltpu.VMEM((L, 4 * hd), BF),
            pltpu.VMEM((2 * L, hd), BF),
            pltpu.VMEM((2 * L, hd), BF),
            pltpu.VMEM((2 * CHUNK, hd), jnp.float32),
        ],
        compiler_params=pltpu.CompilerParams(
            dimension_semantics=("parallel",),
            vmem_limit_bytes=100 * 1024 * 1024,
        ),
    )(x.astype(BF), W_all, b_all)

    o_flat = o_heads.reshape(B * L, D)
    rb = 512 if (B * L) % 512 == 0 else B * L
    y = pl.pallas_call(
        _out_kernel,
        grid=((B * L) // rb,),
        in_specs=[
            pl.BlockSpec((rb, D), lambda i: (i, 0)),
            pl.BlockSpec((D, D), lambda i: (0, 0)),
            pl.BlockSpec((1, D), lambda i: (0, 0)),
        ],
        out_specs=pl.BlockSpec((rb, D), lambda i: (i, 0)),
        out_shape=jax.ShapeDtypeStruct((B * L, D), jnp.float32),
        compiler_params=pltpu.CompilerParams(
            dimension_semantics=("parallel",),
            vmem_limit_bytes=100 * 1024 * 1024,
        ),
    )(o_flat, Wo.astype(BF), bo.reshape(1, D))
    return y.reshape(B, L, D)


# speedup vs baseline: 1.5149x; 1.5149x over previous
import functools

import jax
import jax.numpy as jnp
from jax.experimental import pallas as pl
from jax.experimental.pallas import tpu as pltpu

CHUNK = 64
BF = jnp.bfloat16
G = 4


def _mm(a, b):
    return jax.lax.dot_general(a.astype(BF), b.astype(BF),
                               (((1,), (0,)), ((), ())),
                               preferred_element_type=jnp.float32)


def _mmT(a, b):
    return jax.lax.dot_general(a.astype(BF), b.astype(BF),
                               (((1,), (1,)), ((), ())),
                               preferred_element_type=jnp.float32)


def _mTm(a, b):
    return jax.lax.dot_general(a.astype(BF), b.astype(BF),
                               (((0,), (0,)), ((), ())),
                               preferred_element_type=jnp.float32)


def _delta_kernel(x_ref, w_ref, b_ref, o_ref, scr_ref, cf_ref, bs_ref,
                  s_ref, *, L, D, hd):
    C = CHUNK
    n_chunks = L // C
    w = w_ref[0]
    bias = b_ref[0]
    rb = 512 if L % 512 == 0 else L
    for r in range(L // rb):
        xs = x_ref[0, r * rb:(r + 1) * rb, :]
        scr_ref[r * rb:(r + 1) * rb, :] = (_mmT(xs, w) + bias).astype(BF)

    row = jax.lax.broadcasted_iota(jnp.int32, (C, C), 0)
    col = jax.lax.broadcasted_iota(jnp.int32, (C, C), 1)
    strict = row > col
    incl = row >= col
    eye = jnp.where(row == col, 1.0, 0.0).astype(jnp.float32)
    scale = BF(hd ** -0.5)
    J = tuple(range(G))

    def pre_group(i0):
        pcs = [scr_ref[pl.ds((i0 + j) * C, C), :] for j in J]
        qs = [pc[:, 0:hd] * scale for pc in pcs]
        ks = [pc[:, hd:2 * hd] for pc in pcs]
        betas = [jax.nn.sigmoid(pc[:, 3 * hd:4 * hd].astype(jnp.float32)).astype(BF)
                 for pc in pcs]
        kbs = [ks[j] * betas[j] for j in J]
        vbs = [pcs[j][:, 2 * hd:3 * hd] * betas[j] for j in J]
        ABs = [_mmT(jnp.concatenate([kbs[j], qs[j]], axis=0), ks[j]) for j in J]
        As = [jnp.where(strict, AB[:C], 0.0) for AB in ABs]
        attns = [jnp.where(incl, AB[C:], 0.0) for AB in ABs]
        Xs = [eye - A for A in As]
        Qs = [_mm(A, A) for A in As]
        for _ in range(4):
            RXs = [_mm(Qs[j], Xs[j]) for j in J]
            RQs = [_mm(Qs[j], Qs[j]) for j in J]
            Xs = [Xs[j] + RXs[j] for j in J]
            Qs = RQs
        Xs = [Xs[j] + _mm(Qs[j], Xs[j]) for j in J]
        u0s = [_mm(Xs[j], vbs[j]) for j in J]
        ws = [_mm(Xs[j], kbs[j]) for j in J]
        AUs = [_mm(attns[j], u0s[j]) for j in J]
        AWs = [_mm(attns[j], ws[j]) for j in J]
        KUs = [_mTm(ks[j], u0s[j]) for j in J]
        KWs = [_mTm(ks[j], ws[j]) for j in J]
        for j in J:
            cf = jnp.concatenate([qs[j] - AWs[j], eye - KWs[j]], axis=0)
            bs = jnp.concatenate([AUs[j], KUs[j]], axis=0)
            cf_ref[pl.ds((i0 + j) * 2 * C, 2 * C), :] = cf.astype(BF)
            bs_ref[pl.ds((i0 + j) * 2 * C, 2 * C), :] = bs.astype(BF)

    def scan_chunk(i):
        cf = cf_ref[pl.ds(i * 2 * C, 2 * C), :]
        bs = bs_ref[pl.ds(i * 2 * C, 2 * C), :]
        Z = _mm(cf, s_ref[C:2 * C, :]) + bs.astype(jnp.float32)
        o_ref[0, pl.ds(i * C, C), 0, 0, :] = Z[:C].astype(BF)
        s_ref[...] = Z

    n_groups = n_chunks // G

    s_ref[...] = jnp.zeros((2 * C, hd), jnp.float32)
    pre_group(0)

    def body(jj, carry):
        for j in J:
            scan_chunk(G * jj + j)
        pre_group(G * jj + G)
        return carry

    jax.lax.fori_loop(0, n_groups - 1, body, jnp.float32(0.0))
    for j in J:
        scan_chunk(n_chunks - G + j)


def _out_kernel(o_ref, w_ref, b_ref, y_ref):
    y_ref[...] = _mmT(o_ref[...], w_ref[...]) + b_ref[...]


def kernel(hidden_states, Wq, bq, Wk, bk, Wv, bv, Wb, bb, Wo, bo):
    x = hidden_states
    B, L, D = x.shape
    H = Wb.shape[0]
    hd = D // H
    BH = B * H

    Wq_r = Wq.reshape(H, hd, D)
    Wk_r = Wk.reshape(H, hd, D)
    Wv_r = Wv.reshape(H, hd, D)
    Wb_r = jnp.broadcast_to(Wb[:, None, :], (H, hd, D))
    W_all = jnp.concatenate([Wq_r, Wk_r, Wv_r, Wb_r], axis=1).astype(BF)
    b_all = jnp.concatenate(
        [bq.reshape(H, hd), bk.reshape(H, hd), bv.reshape(H, hd),
         jnp.broadcast_to(bb[:, None], (H, hd))], axis=1).reshape(H, 1, 4 * hd)

    o_heads = pl.pallas_call(
        functools.partial(_delta_kernel, L=L, D=D, hd=hd),
        grid=(BH,),
        in_specs=[
            pl.BlockSpec((1, L, D), lambda i: (i // H, 0, 0)),
            pl.BlockSpec((1, 4 * hd, D), lambda i: (i % H, 0, 0)),
            pl.BlockSpec((1, 1, 4 * hd), lambda i: (i % H, 0, 0)),
        ],
        out_specs=pl.BlockSpec((1, L, 1, 1, hd), lambda i: (i // H, 0, i % H, 0, 0)),
        out_shape=jax.ShapeDtypeStruct((B, L, H, 1, hd), BF),
        scratch_shapes=[
            pltpu.VMEM((L, 4 * hd), BF),
            pltpu.VMEM((2 * L, hd), BF),
            pltpu.VMEM((2 * L, hd), BF),
            pltpu.VMEM((2 * CHUNK, hd), jnp.float32),
        ],
        compiler_params=pltpu.CompilerParams(
            dimension_semantics=("parallel",),
            vmem_limit_bytes=100 * 1024 * 1024,
        ),
    )(x.astype(BF), W_all, b_all)

    o_flat = o_heads.reshape(B * L, D)
    rb = 512 if (B * L) % 512 == 0 else B * L
    y = pl.pallas_call(
        _out_kernel,
        grid=((B * L) // rb,),
        in_specs=[
            pl.BlockSpec((rb, D), lambda i: (i, 0)),
            pl.BlockSpec((D, D), lambda i: (0, 0)),
            pl.BlockSpec((1, D), lambda i: (0, 0)),
        ],
        out_specs=pl.BlockSpec((rb, D), lambda i: (i, 0)),
        out_shape=jax.ShapeDtypeStruct((B * L, D), jnp.float32),
        compiler_params=pltpu.CompilerParams(
            dimension_semantics=("parallel",),
            vmem_limit_bytes=100 * 1024 * 1024,
        ),
    )(o_flat, Wo.astype(BF), bo.reshape(1, D))
    return y.reshape(B, L, D)


# speedup vs baseline: 1.8786x; 1.2401x over previous
import functools

import jax
import jax.numpy as jnp
from jax.experimental import pallas as pl
from jax.experimental.pallas import tpu as pltpu

CHUNK = 64
BF = jnp.bfloat16
G = 8


def _mm(a, b):
    return jax.lax.dot_general(a.astype(BF), b.astype(BF),
                               (((1,), (0,)), ((), ())),
                               preferred_element_type=jnp.float32)


def _mmT(a, b):
    return jax.lax.dot_general(a.astype(BF), b.astype(BF),
                               (((1,), (1,)), ((), ())),
                               preferred_element_type=jnp.float32)


def _mTm(a, b):
    return jax.lax.dot_general(a.astype(BF), b.astype(BF),
                               (((0,), (0,)), ((), ())),
                               preferred_element_type=jnp.float32)


def _delta_kernel(x_ref, w_ref, b_ref, o_ref, scr_ref, cf_ref, bs_ref,
                  s_ref, *, L, D, hd):
    C = CHUNK
    n_chunks = L // C
    w = w_ref[0]
    bias = b_ref[0]
    rb = 512 if L % 512 == 0 else L
    for r in range(L // rb):
        xs = x_ref[0, r * rb:(r + 1) * rb, :]
        scr_ref[r * rb:(r + 1) * rb, :] = (_mmT(xs, w) + bias).astype(BF)

    row = jax.lax.broadcasted_iota(jnp.int32, (C, C), 0)
    col = jax.lax.broadcasted_iota(jnp.int32, (C, C), 1)
    strict = row > col
    incl = row >= col
    eye = jnp.where(row == col, 1.0, 0.0).astype(jnp.float32)
    scale = BF(hd ** -0.5)
    J = tuple(range(G))

    def pre_group(i0):
        pcs = [scr_ref[pl.ds((i0 + j) * C, C), :] for j in J]
        qs = [pc[:, 0:hd] * scale for pc in pcs]
        ks = [pc[:, hd:2 * hd] for pc in pcs]
        betas = [jax.nn.sigmoid(pc[:, 3 * hd:4 * hd].astype(jnp.float32)).astype(BF)
                 for pc in pcs]
        kbs = [ks[j] * betas[j] for j in J]
        vbs = [pcs[j][:, 2 * hd:3 * hd] * betas[j] for j in J]
        ABs = [_mmT(jnp.concatenate([kbs[j], qs[j]], axis=0), ks[j]) for j in J]
        As = [jnp.where(strict, AB[:C], 0.0) for AB in ABs]
        attns = [jnp.where(incl, AB[C:], 0.0) for AB in ABs]
        Xs = [eye - A for A in As]
        Qs = [_mm(A, A) for A in As]
        for _ in range(4):
            RXs = [_mm(Qs[j], Xs[j]) for j in J]
            RQs = [_mm(Qs[j], Qs[j]) for j in J]
            Xs = [Xs[j] + RXs[j] for j in J]
            Qs = RQs
        Xs = [Xs[j] + _mm(Qs[j], Xs[j]) for j in J]
        u0s = [_mm(Xs[j], vbs[j]) for j in J]
        ws = [_mm(Xs[j], kbs[j]) for j in J]
        AUs = [_mm(attns[j], u0s[j]) for j in J]
        AWs = [_mm(attns[j], ws[j]) for j in J]
        KUs = [_mTm(ks[j], u0s[j]) for j in J]
        KWs = [_mTm(ks[j], ws[j]) for j in J]
        for j in J:
            cf = jnp.concatenate([qs[j] - AWs[j], eye - KWs[j]], axis=0)
            bs = jnp.concatenate([AUs[j], KUs[j]], axis=0)
            cf_ref[pl.ds((i0 + j) * 2 * C, 2 * C), :] = cf.astype(BF)
            bs_ref[pl.ds((i0 + j) * 2 * C, 2 * C), :] = bs.astype(BF)

    def scan_chunk(i):
        cf = cf_ref[pl.ds(i * 2 * C, 2 * C), :]
        bs = bs_ref[pl.ds(i * 2 * C, 2 * C), :]
        Z = _mm(cf, s_ref[C:2 * C, :]) + bs.astype(jnp.float32)
        o_ref[0, pl.ds(i * C, C), 0, 0, :] = Z[:C].astype(BF)
        s_ref[...] = Z

    n_groups = n_chunks // G

    s_ref[...] = jnp.zeros((2 * C, hd), jnp.float32)
    pre_group(0)

    def body(jj, carry):
        for j in J:
            scan_chunk(G * jj + j)
        pre_group(G * jj + G)
        return carry

    jax.lax.fori_loop(0, n_groups - 1, body, jnp.float32(0.0))
    for j in J:
        scan_chunk(n_chunks - G + j)


def _out_kernel(o_ref, w_ref, b_ref, y_ref):
    y_ref[...] = _mmT(o_ref[...], w_ref[...]) + b_ref[...]


def kernel(hidden_states, Wq, bq, Wk, bk, Wv, bv, Wb, bb, Wo, bo):
    x = hidden_states
    B, L, D = x.shape
    H = Wb.shape[0]
    hd = D // H
    BH = B * H

    Wq_r = Wq.reshape(H, hd, D)
    Wk_r = Wk.reshape(H, hd, D)
    Wv_r = Wv.reshape(H, hd, D)
    Wb_r = jnp.broadcast_to(Wb[:, None, :], (H, hd, D))
    W_all = jnp.concatenate([Wq_r, Wk_r, Wv_r, Wb_r], axis=1).astype(BF)
    b_all = jnp.concatenate(
        [bq.reshape(H, hd), bk.reshape(H, hd), bv.reshape(H, hd),
         jnp.broadcast_to(bb[:, None], (H, hd))], axis=1).reshape(H, 1, 4 * hd)

    o_heads = pl.pallas_call(
        functools.partial(_delta_kernel, L=L, D=D, hd=hd),
        grid=(BH,),
        in_specs=[
            pl.BlockSpec((1, L, D), lambda i: (i // H, 0, 0)),
            pl.BlockSpec((1, 4 * hd, D), lambda i: (i % H, 0, 0)),
            pl.BlockSpec((1, 1, 4 * hd), lambda i: (i % H, 0, 0)),
        ],
        out_specs=pl.BlockSpec((1, L, 1, 1, hd), lambda i: (i // H, 0, i % H, 0, 0)),
        out_shape=jax.ShapeDtypeStruct((B, L, H, 1, hd), BF),
        scratch_shapes=[
            pltpu.VMEM((L, 4 * hd), BF),
            pltpu.VMEM((2 * L, hd), BF),
            pltpu.VMEM((2 * L, hd), BF),
            pltpu.VMEM((2 * CHUNK, hd), jnp.float32),
        ],
        compiler_params=pltpu.CompilerParams(
            dimension_semantics=("parallel",),
            vmem_limit_bytes=100 * 1024 * 1024,
        ),
    )(x.astype(BF), W_all, b_all)

    o_flat = o_heads.reshape(B * L, D)
    rb = 512 if (B * L) % 512 == 0 else B * L
    y = pl.pallas_call(
        _out_kernel,
        grid=((B * L) // rb,),
        in_specs=[
            pl.BlockSpec((rb, D), lambda i: (i, 0)),
            pl.BlockSpec((D, D), lambda i: (0, 0)),
            pl.BlockSpec((1, D), lambda i: (0, 0)),
        ],
        out_specs=pl.BlockSpec((rb, D), lambda i: (i, 0)),
        out_shape=jax.ShapeDtypeStruct((B * L, D), jnp.float32),
        compiler_params=pltpu.CompilerParams(
            dimension_semantics=("parallel",),
            vmem_limit_bytes=100 * 1024 * 1024,
        ),
    )(o_flat, Wo.astype(BF), bo.reshape(1, D))
    return y.reshape(B, L, D)
